# Initial kernel scaffold; baseline (speedup 1.0000x reference)
import jax
import jax.numpy as jnp
from jax.experimental import pallas as pl


def kernel(x, emb_table, bias):
    raise NotImplementedError("write your pallas kernel here")


# speedup vs baseline: 1.4278x; 1.4278x over previous
import functools

import jax
import jax.numpy as jnp
from jax import lax
from jax.experimental import pallas as pl
from jax.experimental.pallas import tpu as pltpu
from jax.experimental.pallas import tpu_sc as plsc

B = 16384
F = 26
NC = 2
NS = 16
NW = NC * NS
BPW = B // NW
IPW = BPW * F
L = 16


def _body(x_hbm, table_hbm, bias_hbm, out_hbm, idx_v, vals_v, out_v, bias_v, sem):
    wid = lax.axis_index("s") * NC + lax.axis_index("c")
    base = wid * IPW
    pltpu.sync_copy(x_hbm.at[pl.ds(base, IPW)], idx_v)
    pltpu.sync_copy(bias_hbm, bias_v)
    pltpu.async_copy(table_hbm.at[idx_v], vals_v, sem).wait()

    bias_vec = bias_v[...]

    def block(j, _):
        off = j * L
        acc = bias_vec
        for f in range(F):
            acc = acc + vals_v[pl.ds(f * BPW + off, L)]
        out_v[pl.ds(off, L)] = 1.0 / (1.0 + jnp.exp(-acc))
        return 0

    lax.fori_loop(0, BPW // L, block, 0)
    pltpu.sync_copy(out_v, out_hbm.at[pl.ds(wid * BPW, BPW)])


@functools.partial(jax.jit, static_argnames=())
def kernel(x, emb_table, bias):
    x_flat = x.reshape(NW, BPW, F).transpose(0, 2, 1).reshape(-1)
    table_flat = emb_table.reshape(-1)
    bias16 = jnp.broadcast_to(bias, (L,))
    mesh = plsc.VectorSubcoreMesh(core_axis_name="c", subcore_axis_name="s")
    out = pl.kernel(
        _body,
        mesh=mesh,
        out_type=jax.ShapeDtypeStruct((B,), jnp.float32),
        scratch_types=[
            pltpu.VMEM((IPW,), jnp.int32),
            pltpu.VMEM((IPW,), jnp.float32),
            pltpu.VMEM((BPW,), jnp.float32),
            pltpu.VMEM((L,), jnp.float32),
            pltpu.SemaphoreType.DMA,
        ],
    )(x_flat, table_flat, bias16)
    return out.reshape(B, 1)


# speedup vs baseline: 2.3317x; 1.6331x over previous
import functools

import jax
import jax.numpy as jnp
from jax import lax
from jax.experimental import pallas as pl
from jax.experimental.pallas import tpu as pltpu
from jax.experimental.pallas import tpu_sc as plsc

B = 16384
F = 26
NC = 2
NS = 16
NW = NC * NS
BPW = B // NW
IPW = BPW * F
L = 16
NCH = 4
CB = BPW // NCH
CI = CB * F


def _body(x_hbm, table_hbm, bias_hbm, out_hbm, idx_v, vals_v, out_v, bias_v,
          sems):
    wid = lax.axis_index("s") * NC + lax.axis_index("c")
    base = wid * IPW
    pltpu.sync_copy(x_hbm.at[pl.ds(base, IPW)], idx_v)
    pltpu.sync_copy(bias_hbm, bias_v)
    bias_vec = bias_v[...]

    def fire(c):
        return pltpu.async_copy(
            table_hbm.at[idx_v.at[pl.ds(c * CI, CI)]],
            vals_v.at[pl.ds(c * CI, CI)], sems.at[c])

    copies = [fire(0)]
    for c in range(NCH):
        copies[c].wait()
        if c + 1 < NCH:
            copies.append(fire(c + 1))
        cbase = c * CI
        for j in range(CB // L):
            off = j * L
            acc = bias_vec
            for f in range(F):
                acc = acc + vals_v[pl.ds(cbase + f * CB + off, L)]
            out_v[pl.ds(c * CB + off, L)] = 1.0 / (1.0 + jnp.exp(-acc))

    pltpu.sync_copy(out_v, out_hbm.at[pl.ds(wid * BPW, BPW)])


@functools.partial(jax.jit, static_argnames=())
def kernel(x, emb_table, bias):
    x_flat = (x.reshape(NW, NCH, CB, F)
              .transpose(0, 1, 3, 2)
              .reshape(-1))
    table_flat = jnp.pad(emb_table, ((0, 448), (0, 0))).reshape(-1)
    bias16 = jnp.broadcast_to(bias, (L,))
    mesh = plsc.VectorSubcoreMesh(core_axis_name="c", subcore_axis_name="s")
    out = pl.kernel(
        _body,
        mesh=mesh,
        out_type=jax.ShapeDtypeStruct((B,), jnp.float32),
        scratch_types=[
            pltpu.VMEM((IPW,), jnp.int32),
            pltpu.VMEM((IPW,), jnp.float32),
            pltpu.VMEM((BPW,), jnp.float32),
            pltpu.VMEM((L,), jnp.float32),
            pltpu.SemaphoreType.DMA((NCH,)),
        ],
    )(x_flat, table_flat, bias16)
    return out.reshape(B, 1)


# speedup vs baseline: 2.3525x; 1.0089x over previous
import functools

import jax
import jax.numpy as jnp
from jax import lax
from jax.experimental import pallas as pl
from jax.experimental.pallas import tpu as pltpu
from jax.experimental.pallas import tpu_sc as plsc

B = 16384
F = 26
NC = 2
NS = 16
NW = NC * NS
BPW = B // NW
IPW = BPW * F
L = 16
NCH = 4
CB = BPW // NCH
CI = CB * F


def _body(x_hbm, table_hbm, bias_hbm, out_hbm, idx_v, vals_v, out_v, bias_v,
          sems):
    wid = lax.axis_index("s") * NC + lax.axis_index("c")
    base = wid * IPW
    pltpu.sync_copy(x_hbm.at[pl.ds(base, IPW)], idx_v)
    pltpu.sync_copy(bias_hbm, bias_v)
    bias_vec = bias_v[...]

    def fire(c):
        return pltpu.async_copy(
            table_hbm.at[idx_v.at[pl.ds(c * CI, CI)]],
            vals_v.at[pl.ds(c * CI, CI)], sems.at[c])

    copies = [fire(0)]
    for c in range(NCH):
        copies[c].wait()
        if c + 1 < NCH:
            copies.append(fire(c + 1))
        cbase = c * CI

        def block(j, _, cbase=cbase, cob=c * CB):
            off = j * L
            acc = bias_vec
            for f in range(F):
                acc = acc + vals_v[pl.ds(cbase + f * CB + off, L)]
            out_v[pl.ds(cob + off, L)] = 1.0 / (1.0 + jnp.exp(-acc))
            return 0

        lax.fori_loop(0, CB // L, block, 0)

    pltpu.sync_copy(out_v, out_hbm.at[pl.ds(wid * BPW, BPW)])


@functools.partial(jax.jit, static_argnames=())
def kernel(x, emb_table, bias):
    x_flat = (x.reshape(NW, NCH, CB, F)
              .transpose(0, 1, 3, 2)
              .reshape(-1))
    table_flat = jnp.pad(emb_table, ((0, 448), (0, 0))).reshape(-1)
    bias16 = jnp.broadcast_to(bias, (L,))
    mesh = plsc.VectorSubcoreMesh(core_axis_name="c", subcore_axis_name="s")
    out = pl.kernel(
        _body,
        mesh=mesh,
        out_type=jax.ShapeDtypeStruct((B,), jnp.float32),
        scratch_types=[
            pltpu.VMEM((IPW,), jnp.int32),
            pltpu.VMEM((IPW,), jnp.float32),
            pltpu.VMEM((BPW,), jnp.float32),
            pltpu.VMEM((L,), jnp.float32),
            pltpu.SemaphoreType.DMA((NCH,)),
        ],
    )(x_flat, table_flat, bias16)
    return out.reshape(B, 1)


# speedup vs baseline: 2.3775x; 1.0107x over previous
import functools

import jax
import jax.numpy as jnp
from jax import lax
from jax.experimental import pallas as pl
from jax.experimental.pallas import tpu as pltpu
from jax.experimental.pallas import tpu_sc as plsc

B = 16384
F = 26
NC = 2
NS = 16
NW = NC * NS
BPW = B // NW
IPW = BPW * F
L = 16
NCH = 4
CB = BPW // NCH
CI = CB * F


def _body(x_hbm, table_hbm, bias_hbm, out_hbm, idx_v, vals_v, out_v, bias_v,
          sems):
    wid = lax.axis_index("s") * NC + lax.axis_index("c")
    base = wid * IPW
    pltpu.sync_copy(x_hbm.at[pl.ds(base, IPW)], idx_v)
    pltpu.sync_copy(bias_hbm, bias_v)
    bias_vec = bias_v[...]

    pltpu.async_copy(table_hbm.at[idx_v], vals_v, sems).wait()

    def block(j, _):
        off = j * L
        acc = bias_vec
        for f in range(F):
            acc = acc + vals_v[pl.ds(f * BPW + off, L)]
        out_v[pl.ds(off, L)] = 1.0 / (1.0 + jnp.exp(-acc))
        return 0

    lax.fori_loop(0, BPW // L, block, 0)

    pltpu.sync_copy(out_v, out_hbm.at[pl.ds(wid * BPW, BPW)])


@functools.partial(jax.jit, static_argnames=())
def kernel(x, emb_table, bias):
    x_flat = x.reshape(NW, BPW, F).transpose(0, 2, 1).reshape(-1)
    table_flat = jnp.pad(emb_table, ((0, 448), (0, 0))).reshape(-1)
    bias16 = jnp.broadcast_to(bias, (L,))
    mesh = plsc.VectorSubcoreMesh(core_axis_name="c", subcore_axis_name="s")
    out = pl.kernel(
        _body,
        mesh=mesh,
        out_type=jax.ShapeDtypeStruct((B,), jnp.float32),
        scratch_types=[
            pltpu.VMEM((IPW,), jnp.int32),
            pltpu.VMEM((IPW,), jnp.float32),
            pltpu.VMEM((BPW,), jnp.float32),
            pltpu.VMEM((L,), jnp.float32),
            pltpu.SemaphoreType.DMA,
        ],
    )(x_flat, table_flat, bias16)
    return out.reshape(B, 1)
